# Initial kernel scaffold; baseline (speedup 1.0000x reference)
import jax
import jax.numpy as jnp
from jax.experimental import pallas as pl


def kernel(c_pred, v0, vt, t, gen_flag, batch_idx):
    raise NotImplementedError("write your pallas kernel here")


# speedup vs baseline: 1.0590x; 1.0590x over previous
import functools

import jax
import jax.numpy as jnp
from jax import lax
from jax.experimental import pallas as pl
from jax.experimental.pallas import tpu as pltpu
from jax.experimental.pallas import tpu_sc as plsc

N = 1_600_000
C = 32
B = 1024
R = 4000
NW = 32
LANES = 16
UNIT = 128
NUNITS = N // UNIT
BASE_UNITS = NUNITS // NW
EXTRA = NUNITS - BASE_UNITS * NW
SUB = 8192
UNITS_PER_SUB = SUB // UNIT


def _tc_main(c_ref, v0_ref, w_ref, p_ref, lw_ref):
    x = c_ref[...]
    e = jnp.exp(x)
    s = jnp.sum(e, axis=1, keepdims=True)
    p = e * (1.0 / s)
    p_ref[...] = p
    q = jnp.exp(p)
    lse2 = jnp.log(jnp.sum(q, axis=1, keepdims=True))
    oh = lax.broadcasted_iota(jnp.int32, (R, C), 1) == v0_ref[...]
    pv0 = jnp.sum(jnp.where(oh, p, 0.0), axis=1, keepdims=True)
    lw_ref[...] = (lse2 - pv0) * w_ref[...]


def _sc_seg_body(lw_hbm, w_hbm, bidx_hbm, sums_hbm, cnts_hbm,
                 idx_b, lw_b, w_b, idx_t, lw_t, w_t,
                 acc_s, acc_c, out_s, out_c):
    wid = lax.axis_index("s") * 2 + lax.axis_index("c")
    lane_base = lax.iota(jnp.int32, LANES) * B

    def zero_body(i, _):
        z = jnp.zeros((LANES,), jnp.float32)
        acc_s[pl.ds(i * LANES, LANES)] = z
        acc_c[pl.ds(i * LANES, LANES)] = z
        return 0

    lax.fori_loop(0, (LANES * B) // LANES, zero_body, 0)

    u0 = wid * BASE_UNITS + jnp.minimum(wid, EXTRA)
    nu = BASE_UNITS + (wid < EXTRA).astype(jnp.int32)
    e0 = u0 * UNIT
    nf = nu // UNITS_PER_SUB
    rem = nu - nf * UNITS_PER_SUB

    def scat(o, idx_ref, lwv_ref, wv_ref):
        addr = idx_ref[pl.ds(o, LANES)] + lane_base
        plsc.addupdate_scatter(acc_s, [addr], lwv_ref[pl.ds(o, LANES)])
        plsc.addupdate_scatter(acc_c, [addr], wv_ref[pl.ds(o, LANES)])

    def big_body(f, _):
        base = e0 + f * SUB
        pltpu.sync_copy(bidx_hbm.at[pl.ds(base, SUB)], idx_b)
        pltpu.sync_copy(lw_hbm.at[pl.ds(base, SUB)], lw_b)
        pltpu.sync_copy(w_hbm.at[pl.ds(base, SUB)], w_b)

        def vbody(k, _):
            scat(k * LANES, idx_b, lw_b, w_b)
            return 0

        lax.fori_loop(0, SUB // LANES, vbody, 0)
        return 0

    lax.fori_loop(0, nf, big_body, 0)

    def tail_body(tu, _):
        base = e0 + nf * SUB + tu * UNIT
        pltpu.sync_copy(bidx_hbm.at[pl.ds(base, UNIT)], idx_t)
        pltpu.sync_copy(lw_hbm.at[pl.ds(base, UNIT)], lw_t)
        pltpu.sync_copy(w_hbm.at[pl.ds(base, UNIT)], w_t)

        def vbody(k, _):
            scat(k * LANES, idx_t, lw_t, w_t)
            return 0

        lax.fori_loop(0, UNIT // LANES, vbody, 0)
        return 0

    lax.fori_loop(0, rem, tail_body, 0)

    def red_body(cc, _):
        o = cc * LANES
        ssum = jnp.zeros((LANES,), jnp.float32)
        csum = jnp.zeros((LANES,), jnp.float32)
        for l in range(LANES):
            ssum = ssum + acc_s[pl.ds(l * B + o, LANES)]
            csum = csum + acc_c[pl.ds(l * B + o, LANES)]
        out_s[pl.ds(o, LANES)] = ssum
        out_c[pl.ds(o, LANES)] = csum
        return 0

    lax.fori_loop(0, B // LANES, red_body, 0)

    pltpu.sync_copy(out_s, sums_hbm.at[wid])
    pltpu.sync_copy(out_c, cnts_hbm.at[wid])


@functools.cache
def _build_sc_seg():
    mesh = plsc.VectorSubcoreMesh(core_axis_name="c", subcore_axis_name="s")
    return pl.kernel(
        _sc_seg_body,
        mesh=mesh,
        compiler_params=pltpu.CompilerParams(needs_layout_passes=False),
        out_type=[
            jax.ShapeDtypeStruct((NW, B), jnp.float32),
            jax.ShapeDtypeStruct((NW, B), jnp.float32),
        ],
        scratch_types=[
            pltpu.VMEM((SUB,), jnp.int32),
            pltpu.VMEM((SUB,), jnp.float32),
            pltpu.VMEM((SUB,), jnp.float32),
            pltpu.VMEM((UNIT,), jnp.int32),
            pltpu.VMEM((UNIT,), jnp.float32),
            pltpu.VMEM((UNIT,), jnp.float32),
            pltpu.VMEM((LANES * B,), jnp.float32),
            pltpu.VMEM((LANES * B,), jnp.float32),
            pltpu.VMEM((B,), jnp.float32),
            pltpu.VMEM((B,), jnp.float32),
        ],
    )


def _tc_combine(s_ref, c_ref, o_ref):
    s = jnp.sum(s_ref[...], axis=0, keepdims=True)
    c = jnp.sum(c_ref[...], axis=0, keepdims=True)
    loss = jnp.where(c > 0.0, s / jnp.maximum(c, 1.0), 0.0)
    o_ref[...] = jnp.sum(loss, axis=1, keepdims=True) * (1.0 / B)


def kernel(c_pred, v0, vt, t, gen_flag, batch_idx):
    w = gen_flag.astype(jnp.float32)
    p, lw = pl.pallas_call(
        _tc_main,
        grid=(N // R,),
        in_specs=[
            pl.BlockSpec((R, C), lambda i: (i, 0)),
            pl.BlockSpec((R, 1), lambda i: (i, 0)),
            pl.BlockSpec((R, 1), lambda i: (i, 0)),
        ],
        out_specs=[
            pl.BlockSpec((R, C), lambda i: (i, 0)),
            pl.BlockSpec((R, 1), lambda i: (i, 0)),
        ],
        out_shape=[
            jax.ShapeDtypeStruct((N, C), jnp.float32),
            jax.ShapeDtypeStruct((N, 1), jnp.float32),
        ],
    )(c_pred, v0.reshape(N, 1), w.reshape(N, 1))

    sums, cnts = _build_sc_seg()(lw.reshape(N), w, batch_idx.astype(jnp.int32))

    loss_mean = pl.pallas_call(
        _tc_combine,
        out_shape=jax.ShapeDtypeStruct((1, 1), jnp.float32),
    )(sums, cnts)

    return (loss_mean.reshape(()), v0, vt, p, gen_flag)


# speedup vs baseline: 10.1004x; 9.5379x over previous
import functools

import jax
import jax.numpy as jnp
from jax import lax
from jax.experimental import pallas as pl
from jax.experimental.pallas import tpu as pltpu
from jax.experimental.pallas import tpu_sc as plsc

N = 1_600_000
C = 32
B = 1024
RN = 8192
NB = (N + RN - 1) // RN
NW = 32
LANES = 16
UNIT = 128
NUNITS = N // UNIT
BASE_UNITS = NUNITS // NW
EXTRA = NUNITS - BASE_UNITS * NW
SUB = 8192
UNITS_PER_SUB = SUB // UNIT


def _tc_main(c_ref, v0_ref, w_ref, p_ref, lw_ref):
    x = c_ref[...]
    e = jnp.exp(x)
    s = jnp.sum(e, axis=0, keepdims=True)
    p = e * (1.0 / s)
    p_ref[...] = p
    q = jnp.exp(p)
    lse2 = jnp.log(jnp.sum(q, axis=0, keepdims=True))
    oh = lax.broadcasted_iota(jnp.int32, (C, RN), 0) == v0_ref[...].reshape(1, RN)
    pv0 = jnp.sum(jnp.where(oh, p, 0.0), axis=0, keepdims=True)
    lw_ref[...] = ((lse2 - pv0) * w_ref[...].reshape(1, RN)).reshape(RN)


def _sc_seg_body(lw_hbm, w_hbm, bidx_hbm, sums_hbm, cnts_hbm,
                 idx_b, lw_b, w_b, idx_t, lw_t, w_t,
                 acc_s, acc_c, out_s, out_c):
    wid = lax.axis_index("s") * 2 + lax.axis_index("c")
    lane_base = lax.iota(jnp.int32, LANES) * B

    def zero_body(i, _):
        z = jnp.zeros((LANES,), jnp.float32)
        acc_s[pl.ds(i * LANES, LANES)] = z
        acc_c[pl.ds(i * LANES, LANES)] = z
        return 0

    lax.fori_loop(0, (LANES * B) // LANES, zero_body, 0)

    u0 = wid * BASE_UNITS + jnp.minimum(wid, EXTRA)
    nu = BASE_UNITS + (wid < EXTRA).astype(jnp.int32)
    e0 = u0 * UNIT
    nf = nu // UNITS_PER_SUB
    rem = nu - nf * UNITS_PER_SUB

    def scat(o, idx_ref, lwv_ref, wv_ref):
        addr = idx_ref[pl.ds(o, LANES)] + lane_base
        plsc.addupdate_scatter(acc_s, [addr], lwv_ref[pl.ds(o, LANES)])
        plsc.addupdate_scatter(acc_c, [addr], wv_ref[pl.ds(o, LANES)])

    def big_body(f, _):
        base = e0 + f * SUB
        pltpu.sync_copy(bidx_hbm.at[pl.ds(base, SUB)], idx_b)
        pltpu.sync_copy(lw_hbm.at[pl.ds(base, SUB)], lw_b)
        pltpu.sync_copy(w_hbm.at[pl.ds(base, SUB)], w_b)

        def vbody(k, _):
            scat(k * LANES, idx_b, lw_b, w_b)
            return 0

        lax.fori_loop(0, SUB // LANES, vbody, 0)
        return 0

    lax.fori_loop(0, nf, big_body, 0)

    def tail_body(tu, _):
        base = e0 + nf * SUB + tu * UNIT
        pltpu.sync_copy(bidx_hbm.at[pl.ds(base, UNIT)], idx_t)
        pltpu.sync_copy(lw_hbm.at[pl.ds(base, UNIT)], lw_t)
        pltpu.sync_copy(w_hbm.at[pl.ds(base, UNIT)], w_t)

        def vbody(k, _):
            scat(k * LANES, idx_t, lw_t, w_t)
            return 0

        lax.fori_loop(0, UNIT // LANES, vbody, 0)
        return 0

    lax.fori_loop(0, rem, tail_body, 0)

    def red_body(cc, _):
        o = cc * LANES
        ssum = jnp.zeros((LANES,), jnp.float32)
        csum = jnp.zeros((LANES,), jnp.float32)
        for l in range(LANES):
            ssum = ssum + acc_s[pl.ds(l * B + o, LANES)]
            csum = csum + acc_c[pl.ds(l * B + o, LANES)]
        out_s[pl.ds(o, LANES)] = ssum
        out_c[pl.ds(o, LANES)] = csum
        return 0

    lax.fori_loop(0, B // LANES, red_body, 0)

    pltpu.sync_copy(out_s, sums_hbm.at[wid])
    pltpu.sync_copy(out_c, cnts_hbm.at[wid])


@functools.cache
def _build_sc_seg():
    mesh = plsc.VectorSubcoreMesh(core_axis_name="c", subcore_axis_name="s")
    return pl.kernel(
        _sc_seg_body,
        mesh=mesh,
        compiler_params=pltpu.CompilerParams(needs_layout_passes=False),
        out_type=[
            jax.ShapeDtypeStruct((NW, B), jnp.float32),
            jax.ShapeDtypeStruct((NW, B), jnp.float32),
        ],
        scratch_types=[
            pltpu.VMEM((SUB,), jnp.int32),
            pltpu.VMEM((SUB,), jnp.float32),
            pltpu.VMEM((SUB,), jnp.float32),
            pltpu.VMEM((UNIT,), jnp.int32),
            pltpu.VMEM((UNIT,), jnp.float32),
            pltpu.VMEM((UNIT,), jnp.float32),
            pltpu.VMEM((LANES * B,), jnp.float32),
            pltpu.VMEM((LANES * B,), jnp.float32),
            pltpu.VMEM((B,), jnp.float32),
            pltpu.VMEM((B,), jnp.float32),
        ],
    )


def _tc_combine(s_ref, c_ref, o_ref):
    s = jnp.sum(s_ref[...], axis=0, keepdims=True)
    c = jnp.sum(c_ref[...], axis=0, keepdims=True)
    loss = jnp.where(c > 0.0, s / jnp.maximum(c, 1.0), 0.0)
    o_ref[...] = jnp.sum(loss, axis=1, keepdims=True) * (1.0 / B)


def kernel(c_pred, v0, vt, t, gen_flag, batch_idx):
    w = gen_flag.astype(jnp.float32)
    ct = c_pred.T
    pt, lw = pl.pallas_call(
        _tc_main,
        grid=(NB,),
        in_specs=[
            pl.BlockSpec((C, RN), lambda i: (0, i)),
            pl.BlockSpec((RN,), lambda i: (i,)),
            pl.BlockSpec((RN,), lambda i: (i,)),
        ],
        out_specs=[
            pl.BlockSpec((C, RN), lambda i: (0, i)),
            pl.BlockSpec((RN,), lambda i: (i,)),
        ],
        out_shape=[
            jax.ShapeDtypeStruct((C, N), jnp.float32),
            jax.ShapeDtypeStruct((N,), jnp.float32),
        ],
    )(ct, v0, w)
    p = pt.T

    sums, cnts = _build_sc_seg()(lw, w, batch_idx.astype(jnp.int32))

    loss_mean = pl.pallas_call(
        _tc_combine,
        out_shape=jax.ShapeDtypeStruct((1, 1), jnp.float32),
    )(sums, cnts)

    return (loss_mean.reshape(()), v0, vt, p, gen_flag)


# speedup vs baseline: 10.5624x; 1.0457x over previous
import functools

import jax
import jax.numpy as jnp
from jax import lax
from jax.experimental import pallas as pl
from jax.experimental.pallas import tpu as pltpu
from jax.experimental.pallas import tpu_sc as plsc

N = 1_600_000
C = 32
B = 1024
RN = 8192
NB = (N + RN - 1) // RN
NW = 32
LANES = 16
UNIT = 128
NUNITS = N // UNIT
BASE_UNITS = NUNITS // NW
EXTRA = NUNITS - BASE_UNITS * NW
SUB = 8192
UNITS_PER_SUB = SUB // UNIT


def _tc_main(c_ref, v0_ref, w_ref, p_ref, lw_ref):
    x = c_ref[...]
    e = jnp.exp(x)
    s = jnp.sum(e, axis=0, keepdims=True)
    p = e * (1.0 / s)
    p_ref[...] = p
    q = jnp.exp(p)
    lse2 = jnp.log(jnp.sum(q, axis=0, keepdims=True))
    oh = lax.broadcasted_iota(jnp.int32, (C, RN), 0) == v0_ref[...].reshape(1, RN)
    pv0 = jnp.sum(jnp.where(oh, p, 0.0), axis=0, keepdims=True)
    lw_ref[...] = ((lse2 - pv0) * w_ref[...].reshape(1, RN)).reshape(RN)


NF = BASE_UNITS // UNITS_PER_SUB
UNROLL = 8


def _sc_seg_body(lw_hbm, w_hbm, bidx_hbm, sums_hbm, cnts_hbm,
                 idx_b0, lw_b0, w_b0, idx_b1, lw_b1, w_b1,
                 idx_t, lw_t, w_t,
                 acc_s, acc_c, out_s, out_c, sem0, sem1):
    wid = lax.axis_index("s") * 2 + lax.axis_index("c")
    lane_base = lax.iota(jnp.int32, LANES) * B

    def zero_body(i, _):
        z = jnp.zeros((LANES,), jnp.float32)
        for u in range(UNROLL):
            acc_s[pl.ds((i * UNROLL + u) * LANES, LANES)] = z
            acc_c[pl.ds((i * UNROLL + u) * LANES, LANES)] = z
        return 0

    lax.fori_loop(0, (LANES * B) // (LANES * UNROLL), zero_body, 0)

    u0 = wid * BASE_UNITS + jnp.minimum(wid, EXTRA)
    nu = BASE_UNITS + (wid < EXTRA).astype(jnp.int32)
    e0 = u0 * UNIT
    rem = nu - NF * UNITS_PER_SUB

    bufs = [(idx_b0, lw_b0, w_b0), (idx_b1, lw_b1, w_b1)]
    sems = [sem0, sem1]

    def issue(setidx, chunk):
        base = e0 + chunk * SUB
        ib, lb, wb = bufs[setidx]
        return [
            pltpu.async_copy(bidx_hbm.at[pl.ds(base, SUB)], ib, sems[setidx]),
            pltpu.async_copy(lw_hbm.at[pl.ds(base, SUB)], lb, sems[setidx]),
            pltpu.async_copy(w_hbm.at[pl.ds(base, SUB)], wb, sems[setidx]),
        ]

    def scat(o, idx_ref, lwv_ref, wv_ref):
        addr = idx_ref[pl.ds(o, LANES)] + lane_base
        plsc.addupdate_scatter(acc_s, [addr], lwv_ref[pl.ds(o, LANES)])
        plsc.addupdate_scatter(acc_c, [addr], wv_ref[pl.ds(o, LANES)])

    pending = issue(0, 0)
    for f in range(NF):
        cur = f % 2
        if f + 1 < NF:
            nxt = issue(1 - cur, f + 1)
        else:
            nxt = None
        for h in pending:
            h.wait()
        pending = nxt
        ib, lb, wb = bufs[cur]

        def vbody(k, _):
            for u in range(UNROLL):
                scat((k * UNROLL + u) * LANES, ib, lb, wb)
            return 0

        lax.fori_loop(0, SUB // (LANES * UNROLL), vbody, 0)

    def tail_body(tu, _):
        base = e0 + NF * SUB + tu * UNIT
        pltpu.sync_copy(bidx_hbm.at[pl.ds(base, UNIT)], idx_t)
        pltpu.sync_copy(lw_hbm.at[pl.ds(base, UNIT)], lw_t)
        pltpu.sync_copy(w_hbm.at[pl.ds(base, UNIT)], w_t)

        def vbody(k, _):
            scat(k * LANES, idx_t, lw_t, w_t)
            return 0

        lax.fori_loop(0, UNIT // LANES, vbody, 0)
        return 0

    lax.fori_loop(0, rem, tail_body, 0)

    def red_body(cc, _):
        o = cc * LANES
        ssum = jnp.zeros((LANES,), jnp.float32)
        csum = jnp.zeros((LANES,), jnp.float32)
        for l in range(LANES):
            ssum = ssum + acc_s[pl.ds(l * B + o, LANES)]
            csum = csum + acc_c[pl.ds(l * B + o, LANES)]
        out_s[pl.ds(o, LANES)] = ssum
        out_c[pl.ds(o, LANES)] = csum
        return 0

    lax.fori_loop(0, B // LANES, red_body, 0)

    pltpu.sync_copy(out_s, sums_hbm.at[wid])
    pltpu.sync_copy(out_c, cnts_hbm.at[wid])


@functools.cache
def _build_sc_seg():
    mesh = plsc.VectorSubcoreMesh(core_axis_name="c", subcore_axis_name="s")
    return pl.kernel(
        _sc_seg_body,
        mesh=mesh,
        compiler_params=pltpu.CompilerParams(needs_layout_passes=False),
        out_type=[
            jax.ShapeDtypeStruct((NW, B), jnp.float32),
            jax.ShapeDtypeStruct((NW, B), jnp.float32),
        ],
        scratch_types=[
            pltpu.VMEM((SUB,), jnp.int32),
            pltpu.VMEM((SUB,), jnp.float32),
            pltpu.VMEM((SUB,), jnp.float32),
            pltpu.VMEM((SUB,), jnp.int32),
            pltpu.VMEM((SUB,), jnp.float32),
            pltpu.VMEM((SUB,), jnp.float32),
            pltpu.VMEM((UNIT,), jnp.int32),
            pltpu.VMEM((UNIT,), jnp.float32),
            pltpu.VMEM((UNIT,), jnp.float32),
            pltpu.VMEM((LANES * B,), jnp.float32),
            pltpu.VMEM((LANES * B,), jnp.float32),
            pltpu.VMEM((B,), jnp.float32),
            pltpu.VMEM((B,), jnp.float32),
            pltpu.SemaphoreType.DMA,
            pltpu.SemaphoreType.DMA,
        ],
    )


def _tc_combine(s_ref, c_ref, o_ref):
    s = jnp.sum(s_ref[...], axis=0, keepdims=True)
    c = jnp.sum(c_ref[...], axis=0, keepdims=True)
    loss = jnp.where(c > 0.0, s / jnp.maximum(c, 1.0), 0.0)
    o_ref[...] = jnp.sum(loss, axis=1, keepdims=True) * (1.0 / B)


def kernel(c_pred, v0, vt, t, gen_flag, batch_idx):
    w = gen_flag.astype(jnp.float32)
    ct = c_pred.T
    pt, lw = pl.pallas_call(
        _tc_main,
        grid=(NB,),
        in_specs=[
            pl.BlockSpec((C, RN), lambda i: (0, i)),
            pl.BlockSpec((RN,), lambda i: (i,)),
            pl.BlockSpec((RN,), lambda i: (i,)),
        ],
        out_specs=[
            pl.BlockSpec((C, RN), lambda i: (0, i)),
            pl.BlockSpec((RN,), lambda i: (i,)),
        ],
        out_shape=[
            jax.ShapeDtypeStruct((C, N), jnp.float32),
            jax.ShapeDtypeStruct((N,), jnp.float32),
        ],
    )(ct, v0, w)
    p = pt.T

    sums, cnts = _build_sc_seg()(lw, w, batch_idx.astype(jnp.int32))

    loss_mean = pl.pallas_call(
        _tc_combine,
        out_shape=jax.ShapeDtypeStruct((1, 1), jnp.float32),
    )(sums, cnts)

    return (loss_mean.reshape(()), v0, vt, p, gen_flag)


# speedup vs baseline: 10.8424x; 1.0265x over previous
import functools

import jax
import jax.numpy as jnp
from jax import lax
from jax.experimental import pallas as pl
from jax.experimental.pallas import tpu as pltpu
from jax.experimental.pallas import tpu_sc as plsc

N = 1_600_000
C = 32
B = 1024
RN = 8192
NB = (N + RN - 1) // RN
NW = 32
LANES = 16
UNIT = 128
NUNITS = N // UNIT
BASE_UNITS = NUNITS // NW
EXTRA = NUNITS - BASE_UNITS * NW
SUB = 16384
UNITS_PER_SUB = SUB // UNIT
NF = BASE_UNITS // UNITS_PER_SUB
UNROLL = 8


def _tc_main(c_ref, v0_ref, w_ref, bi_ref, p_ref, pk_ref):
    x = c_ref[...]
    e = jnp.exp(x)
    s = jnp.sum(e, axis=0, keepdims=True)
    p = e * (1.0 / s)
    p_ref[...] = p
    q = jnp.exp(p)
    lse2 = jnp.log(jnp.sum(q, axis=0, keepdims=True))
    oh = lax.broadcasted_iota(jnp.int32, (C, RN), 0) == v0_ref[...].reshape(1, RN)
    pv0 = jnp.sum(jnp.where(oh, p, 0.0), axis=0, keepdims=True)
    lw = ((lse2 - pv0) * w_ref[...].reshape(1, RN)).reshape(RN)
    bits = lax.bitcast_convert_type(lw, jnp.uint32) + jnp.uint32(0x8000)
    pk_ref[...] = (bi_ref[...].astype(jnp.uint32) << 16) | (bits >> 16)


def _sc_seg_body(pk_hbm, sums_hbm, cnts_hbm,
                 pk_b0, pk_b1, pk_t, acc_s, acc_c, out_s, out_c, sem0, sem1):
    wid = lax.axis_index("s") * 2 + lax.axis_index("c")
    lane_base = lax.iota(jnp.int32, LANES) * B
    one = jnp.full((LANES,), 1.0, jnp.float32)
    zero = jnp.zeros((LANES,), jnp.float32)

    def zero_body(i, _):
        for u in range(UNROLL):
            acc_s[pl.ds((i * UNROLL + u) * LANES, LANES)] = zero
            acc_c[pl.ds((i * UNROLL + u) * LANES, LANES)] = zero
        return 0

    lax.fori_loop(0, (LANES * B) // (LANES * UNROLL), zero_body, 0)

    u0 = wid * BASE_UNITS + jnp.minimum(wid, EXTRA)
    nu = BASE_UNITS + (wid < EXTRA).astype(jnp.int32)
    e0 = u0 * UNIT
    rem = nu - NF * UNITS_PER_SUB

    bufs = [pk_b0, pk_b1]
    sems = [sem0, sem1]

    def scat(o, pk_ref):
        v = pk_ref[pl.ds(o, LANES)]
        idx = lax.convert_element_type(v >> 16, jnp.int32)
        lwv = plsc.bitcast(v << 16, jnp.float32)
        cv = jnp.where(lwv != 0.0, one, zero)
        addr = idx + lane_base
        plsc.addupdate_scatter(acc_s, [addr], lwv)
        plsc.addupdate_scatter(acc_c, [addr], cv)

    pending = pltpu.async_copy(pk_hbm.at[pl.ds(e0, SUB)], bufs[0], sems[0])
    for f in range(NF):
        cur = f % 2
        nxt = None
        if f + 1 < NF:
            nxt = pltpu.async_copy(
                pk_hbm.at[pl.ds(e0 + (f + 1) * SUB, SUB)], bufs[1 - cur],
                sems[1 - cur])
        pending.wait()
        pending = nxt
        pk_ref = bufs[cur]

        def vbody(k, _):
            for u in range(UNROLL):
                scat((k * UNROLL + u) * LANES, pk_ref)
            return 0

        lax.fori_loop(0, SUB // (LANES * UNROLL), vbody, 0)

    def tail_body(tu, _):
        base = e0 + NF * SUB + tu * UNIT
        pltpu.sync_copy(pk_hbm.at[pl.ds(base, UNIT)], pk_t)

        def vbody(k, _):
            scat(k * LANES, pk_t)
            return 0

        lax.fori_loop(0, UNIT // LANES, vbody, 0)
        return 0

    lax.fori_loop(0, rem, tail_body, 0)

    def red_body(cc, _):
        o = cc * LANES
        ssum = jnp.zeros((LANES,), jnp.float32)
        csum = jnp.zeros((LANES,), jnp.float32)
        for l in range(LANES):
            ssum = ssum + acc_s[pl.ds(l * B + o, LANES)]
            csum = csum + acc_c[pl.ds(l * B + o, LANES)]
        out_s[pl.ds(o, LANES)] = ssum
        out_c[pl.ds(o, LANES)] = csum
        return 0

    lax.fori_loop(0, B // LANES, red_body, 0)

    pltpu.sync_copy(out_s, sums_hbm.at[wid])
    pltpu.sync_copy(out_c, cnts_hbm.at[wid])


@functools.cache
def _build_sc_seg():
    mesh = plsc.VectorSubcoreMesh(core_axis_name="c", subcore_axis_name="s")
    return pl.kernel(
        _sc_seg_body,
        mesh=mesh,
        compiler_params=pltpu.CompilerParams(needs_layout_passes=False),
        out_type=[
            jax.ShapeDtypeStruct((NW, B), jnp.float32),
            jax.ShapeDtypeStruct((NW, B), jnp.float32),
        ],
        scratch_types=[
            pltpu.VMEM((SUB,), jnp.uint32),
            pltpu.VMEM((SUB,), jnp.uint32),
            pltpu.VMEM((UNIT,), jnp.uint32),
            pltpu.VMEM((LANES * B,), jnp.float32),
            pltpu.VMEM((LANES * B,), jnp.float32),
            pltpu.VMEM((B,), jnp.float32),
            pltpu.VMEM((B,), jnp.float32),
            pltpu.SemaphoreType.DMA,
            pltpu.SemaphoreType.DMA,
        ],
    )


def _tc_combine(s_ref, c_ref, o_ref):
    s = jnp.sum(s_ref[...], axis=0, keepdims=True)
    c = jnp.sum(c_ref[...], axis=0, keepdims=True)
    loss = jnp.where(c > 0.0, s / jnp.maximum(c, 1.0), 0.0)
    o_ref[...] = jnp.sum(loss, axis=1, keepdims=True) * (1.0 / B)


def kernel(c_pred, v0, vt, t, gen_flag, batch_idx):
    w = gen_flag.astype(jnp.float32)
    ct = c_pred.T
    pt, pk = pl.pallas_call(
        _tc_main,
        grid=(NB,),
        in_specs=[
            pl.BlockSpec((C, RN), lambda i: (0, i)),
            pl.BlockSpec((RN,), lambda i: (i,)),
            pl.BlockSpec((RN,), lambda i: (i,)),
            pl.BlockSpec((RN,), lambda i: (i,)),
        ],
        out_specs=[
            pl.BlockSpec((C, RN), lambda i: (0, i)),
            pl.BlockSpec((RN,), lambda i: (i,)),
        ],
        out_shape=[
            jax.ShapeDtypeStruct((C, N), jnp.float32),
            jax.ShapeDtypeStruct((N,), jnp.uint32),
        ],
    )(ct, v0, w, batch_idx.astype(jnp.int32))
    p = pt.T

    sums, cnts = _build_sc_seg()(pk)

    loss_mean = pl.pallas_call(
        _tc_combine,
        out_shape=jax.ShapeDtypeStruct((1, 1), jnp.float32),
    )(sums, cnts)

    return (loss_mean.reshape(()), v0, vt, p, gen_flag)
